# baseline (speedup 1.0000x reference)
import jax
import jax.numpy as jnp
from jax.experimental import pallas as pl
from jax.experimental.pallas import tpu as pltpu

_Y_CHUNK = 16384


def _dotn(a, b):
    return jax.lax.dot_general(a, b, (((1,), (1,)), ((), ())),
                               preferred_element_type=jnp.float32)


def _dn_step(x_ref, w_ref, age_ref, y2z_ref, out_ref, max_ref, cand_ref, ssq_ref):
    i = pl.program_id(0)
    nsteps = pl.num_programs(0)

    @pl.when(i == 0)
    def _init():
        max_ref[...] = jnp.full_like(max_ref, -jnp.inf)
        cand_ref[...] = jnp.zeros_like(cand_ref)
        ssq_ref[...] = jnp.zeros_like(ssq_ref)

    xb = x_ref[...]
    w = w_ref[...]
    y2z = y2z_ref[...]
    age = age_ref[...]
    chunk = w.shape[0]

    rssq = jnp.sum(w * w, axis=1, keepdims=True)
    wn = w * (1.0 / jnp.maximum(jnp.sqrt(rssq), 1e-12))
    dots = _dotn(xb, wn.astype(jnp.bfloat16))

    act = jnp.where(age >= 1.0, 1.0, 0.0)
    y_pre = dots * act

    local_max = jnp.max(y_pre, axis=1, keepdims=True)
    iota = jax.lax.broadcasted_iota(jnp.int32, y_pre.shape, 1)
    eq = y_pre == local_max
    first = jnp.min(jnp.where(eq, iota, chunk), axis=1, keepdims=True)
    onehot = (iota == first).astype(jnp.float32)

    cand = _dotn(onehot, y2z)
    better = local_max > max_ref[...]
    max_ref[...] = jnp.where(better, local_max, max_ref[...])
    cand_ref[...] = jnp.where(better, cand, cand_ref[...])

    ones_y = jnp.ones((1, chunk), jnp.float32)
    ssq_ref[...] += _dotn(ones_y, y2z * y2z)

    @pl.when(i == nsteps - 1)
    def _fin():
        zn = jnp.maximum(jnp.sqrt(ssq_ref[...]), 1e-12)
        out_ref[...] = cand_ref[...] / zn


def kernel(x, z, per_item, x2y_w, z2y_w, y2z_w, y_neuron_age):
    batch = x.shape[0]
    xf = x.reshape(batch, -1)
    x_dim = xf.shape[1]
    y_num = x2y_w.shape[0]
    z_num = y2z_w.shape[0]
    grid = y_num // _Y_CHUNK

    xn = jnp.sqrt(jnp.sum(xf * xf, axis=1, keepdims=True))
    xfn = xf / jnp.maximum(xn, 1e-12)
    xb = xfn.astype(jnp.bfloat16)

    return pl.pallas_call(
        _dn_step,
        grid=(grid,),
        in_specs=[
            pl.BlockSpec((batch, x_dim), lambda i: (0, 0)),
            pl.BlockSpec((_Y_CHUNK, x_dim), lambda i: (i, 0)),
            pl.BlockSpec((1, _Y_CHUNK), lambda i: (0, i)),
            pl.BlockSpec((z_num, _Y_CHUNK), lambda i: (0, i)),
        ],
        out_specs=pl.BlockSpec((batch, z_num), lambda i: (0, 0)),
        out_shape=jax.ShapeDtypeStruct((batch, z_num), jnp.float32),
        scratch_shapes=[
            pltpu.VMEM((batch, 1), jnp.float32),
            pltpu.VMEM((batch, z_num), jnp.float32),
            pltpu.VMEM((1, z_num), jnp.float32),
        ],
    )(xb, x2y_w, y_neuron_age, y2z_w)


# speedup vs baseline: 1.1356x; 1.1356x over previous
import jax
import jax.numpy as jnp
from jax.experimental import pallas as pl
from jax.experimental.pallas import tpu as pltpu

_Y_CHUNK = 8192


def _dotn(a, b):
    return jax.lax.dot_general(a, b, (((1,), (1,)), ((), ())),
                               preferred_element_type=jnp.float32)


def _dn_step(x_ref, w_ref, age_ref, y2z_ref, out_ref, max_ref, cand_ref, ssq_ref):
    i = pl.program_id(0)
    nsteps = pl.num_programs(0)

    @pl.when(i == 0)
    def _init():
        max_ref[...] = jnp.full_like(max_ref, -jnp.inf)
        cand_ref[...] = jnp.zeros_like(cand_ref)
        ssq_ref[...] = jnp.zeros_like(ssq_ref)

    xb = x_ref[...]
    w = w_ref[...]
    y2z = y2z_ref[...]
    age = age_ref[...]
    chunk = w.shape[0]

    rssq = jnp.sum(w * w, axis=1, keepdims=True)
    wn = w * (1.0 / jnp.maximum(jnp.sqrt(rssq), 1e-12))
    dots = _dotn(xb, wn.astype(jnp.bfloat16))

    act = jnp.where(age >= 1.0, 1.0, 0.0)
    y_pre = dots * act

    local_max = jnp.max(y_pre, axis=1, keepdims=True)
    iota = jax.lax.broadcasted_iota(jnp.int32, y_pre.shape, 1)
    eq = y_pre == local_max
    first = jnp.min(jnp.where(eq, iota, chunk), axis=1, keepdims=True)
    onehot = (iota == first).astype(jnp.float32)

    cand = _dotn(onehot, y2z)
    better = local_max > max_ref[...]
    max_ref[...] = jnp.where(better, local_max, max_ref[...])
    cand_ref[...] = jnp.where(better, cand, cand_ref[...])

    ones_y = jnp.ones((1, chunk), jnp.float32)
    ssq_ref[...] += _dotn(ones_y, y2z * y2z)

    @pl.when(i == nsteps - 1)
    def _fin():
        zn = jnp.maximum(jnp.sqrt(ssq_ref[...]), 1e-12)
        out_ref[...] = cand_ref[...] / zn


def kernel(x, z, per_item, x2y_w, z2y_w, y2z_w, y_neuron_age):
    batch = x.shape[0]
    xf = x.reshape(batch, -1)
    x_dim = xf.shape[1]
    y_num = x2y_w.shape[0]
    z_num = y2z_w.shape[0]
    grid = y_num // _Y_CHUNK

    xn = jnp.sqrt(jnp.sum(xf * xf, axis=1, keepdims=True))
    xfn = xf / jnp.maximum(xn, 1e-12)
    xb = xfn.astype(jnp.bfloat16)

    return pl.pallas_call(
        _dn_step,
        grid=(grid,),
        in_specs=[
            pl.BlockSpec((batch, x_dim), lambda i: (0, 0)),
            pl.BlockSpec((_Y_CHUNK, x_dim), lambda i: (i, 0)),
            pl.BlockSpec((1, _Y_CHUNK), lambda i: (0, i)),
            pl.BlockSpec((z_num, _Y_CHUNK), lambda i: (0, i)),
        ],
        out_specs=pl.BlockSpec((batch, z_num), lambda i: (0, 0)),
        out_shape=jax.ShapeDtypeStruct((batch, z_num), jnp.float32),
        scratch_shapes=[
            pltpu.VMEM((batch, 1), jnp.float32),
            pltpu.VMEM((batch, z_num), jnp.float32),
            pltpu.VMEM((1, z_num), jnp.float32),
        ],
    )(xb, x2y_w, y_neuron_age, y2z_w)
